# baseline (speedup 1.0000x reference)
import functools

import jax
import jax.numpy as jnp
from jax import lax
from jax.experimental import pallas as pl
from jax.experimental.pallas import tpu as pltpu
from jax.experimental.pallas import tpu_sc as plsc

_L = 16


def _sigmoid(x):
    return 1.0 / (1.0 + jnp.exp(-x))


@functools.partial(jax.jit, static_argnums=(4,))
def _run(table, idx_t, dns_t, wb, num_workers):
    vpad = table.shape[0]
    num_fields, B = idx_t.shape
    ndense = dns_t.shape[0]
    bpw = B // num_workers
    ngroups = bpw // _L
    mesh = plsc.VectorSubcoreMesh(core_axis_name="c", subcore_axis_name="s")

    @functools.partial(
        pl.kernel,
        mesh=mesh,
        out_type=jax.ShapeDtypeStruct((B,), jnp.float32),
        scratch_types=[
            pltpu.VMEM((vpad,), jnp.float32),
            pltpu.VMEM((num_fields, bpw), jnp.int32),
            pltpu.VMEM((ndense, bpw), jnp.float32),
            pltpu.VMEM((_L,), jnp.float32),
            pltpu.VMEM((bpw,), jnp.float32),
            pltpu.SemaphoreType.DMA,
        ],
        compiler_params=pltpu.CompilerParams(
            needs_layout_passes=False, use_tc_tiling_on_sc=True
        ),
    )
    def k(table_hbm, idx_hbm, dns_hbm, wb_hbm, out_hbm,
          table_v, idx_v, dns_v, wb_v, out_v, sem):
        wid = lax.axis_index("s") * 2 + lax.axis_index("c")
        base = wid * bpw
        c1 = pltpu.async_copy(table_hbm, table_v, sem)
        c2 = pltpu.async_copy(idx_hbm.at[:, pl.ds(base, bpw)], idx_v, sem)
        c3 = pltpu.async_copy(dns_hbm.at[:, pl.ds(base, bpw)], dns_v, sem)
        c4 = pltpu.async_copy(wb_hbm, wb_v, sem)
        c1.wait()
        c2.wait()
        c3.wait()
        c4.wait()

        lane = lax.iota(jnp.int32, _L)
        zero = jnp.zeros((_L,), jnp.int32)
        wvecs = [plsc.load_gather(wb_v, [zero + j]) for j in range(ndense)]
        bvec = plsc.load_gather(wb_v, [zero + ndense])

        def body(g, _):
            rows = g * _L + lane
            acc = bvec
            for j in range(ndense):
                dv = plsc.load_gather(dns_v, [zero + j, rows])
                acc = acc + dv * wvecs[j]
            for f in range(num_fields):
                ii = plsc.load_gather(idx_v, [zero + f, rows])
                acc = acc + plsc.load_gather(table_v, [ii])
            out_v[pl.ds(g * _L, _L)] = _sigmoid(acc)
            return _

        lax.fori_loop(0, ngroups, body, None)
        pltpu.sync_copy(out_v, out_hbm.at[pl.ds(base, bpw)])

    return k(table, idx_t, dns_t, wb)


def kernel(dense_input, sparse_input, emb_table, fm_W, fm_b):
    B, ndense = dense_input.shape
    V = emb_table.shape[0]
    NW = 32

    idx_t = sparse_input.astype(jnp.int32).T
    dns_t = dense_input.astype(jnp.float32).T
    vpad = ((V + 127) // 128) * 128
    table = jnp.zeros((vpad,), jnp.float32).at[:V].set(emb_table[:, 0])
    wb = jnp.zeros((_L,), jnp.float32)
    wb = wb.at[:ndense].set(fm_W.reshape(ndense).astype(jnp.float32))
    wb = wb.at[ndense].set(fm_b.reshape(())[...].astype(jnp.float32))

    out = _run(table, idx_t, dns_t, wb, NW)
    return out.reshape(B, 1)
